# baseline (speedup 1.0000x reference)
import functools

import jax
import jax.numpy as jnp
from jax import lax
from jax.experimental import pallas as pl
from jax.experimental.pallas import tpu as pltpu
from jax.experimental.pallas import tpu_sc as plsc

BATCH = 4096
NUM_FIELDS = 26
NC = 2
NS = 16
LANES = 16
NW = NC * NS
ROWS_PER_W = BATCH // NW


def _sc_kernel(xt_hbm, table_hbm, bias_hbm, out_hbm, idx_v, vals_v, out_v,
               bias_v, sem):
    wid = lax.axis_index("s") * NC + lax.axis_index("c")
    base = wid * ROWS_PER_W

    stage = [
        pltpu.async_copy(
            xt_hbm.at[f, pl.ds(base, ROWS_PER_W)],
            idx_v.at[pl.ds(f * ROWS_PER_W, ROWS_PER_W)],
            sem,
        )
        for f in range(NUM_FIELDS)
    ]
    bias_cp = pltpu.async_copy(bias_hbm, bias_v, sem)

    groups = (7, 7, 6, 6)
    bounds = []
    lo = 0
    for g in groups:
        bounds.append((lo, g))
        lo += g
    gathers = []
    done = 0
    for (glo, gn) in bounds:
        for f in range(glo, glo + gn):
            stage[f].wait()
        n = gn * ROWS_PER_W
        gathers.append(
            pltpu.async_copy(
                table_hbm.at[idx_v.at[pl.ds(glo * ROWS_PER_W, n)]],
                vals_v.at[pl.ds(glo * ROWS_PER_W, n)],
                sem,
            ))

    bias_cp.wait()
    bias_vec = plsc.load_gather(bias_v, [jnp.zeros((LANES,), jnp.int32)])

    nchunk = ROWS_PER_W // LANES
    accs = [bias_vec] * nchunk
    for gi, (glo, gn) in enumerate(bounds):
        gathers[gi].wait()
        for j in range(nchunk):
            acc = accs[j]
            for f in range(glo, glo + gn):
                acc = acc + vals_v[pl.ds(f * ROWS_PER_W + j * LANES, LANES)]
            accs[j] = acc
    for j in range(nchunk):
        out_v[pl.ds(j * LANES, LANES)] = accs[j]

    pltpu.sync_copy(out_v, out_hbm.at[pl.ds(base, ROWS_PER_W)])


@jax.jit
def _run(xt, table_flat, bias):
    mesh = plsc.VectorSubcoreMesh(
        core_axis_name="c", subcore_axis_name="s",
        num_cores=NC, num_subcores=NS)
    f = functools.partial(
        pl.kernel,
        out_type=jax.ShapeDtypeStruct((BATCH,), jnp.float32),
        mesh=mesh,
        scratch_types=[
            pltpu.VMEM((NUM_FIELDS * ROWS_PER_W,), jnp.int32),
            pltpu.VMEM((NUM_FIELDS * ROWS_PER_W,), jnp.float32),
            pltpu.VMEM((ROWS_PER_W,), jnp.float32),
            pltpu.VMEM((1,), jnp.float32),
            pltpu.SemaphoreType.DMA,
        ],
        compiler_params=pltpu.CompilerParams(needs_layout_passes=False),
    )(_sc_kernel)
    return f(xt, table_flat, bias)


def kernel(x, table, bias):
    xt = x.astype(jnp.int32).T
    table_flat = table.reshape(-1)
    out = _run(xt, table_flat, bias.astype(jnp.float32))
    return out.reshape(BATCH, 1)


# speedup vs baseline: 1.1385x; 1.1385x over previous
import functools

import jax
import jax.numpy as jnp
from jax import lax
from jax.experimental import pallas as pl
from jax.experimental.pallas import tpu as pltpu
from jax.experimental.pallas import tpu_sc as plsc

BATCH = 4096
NUM_FIELDS = 26
NC = 2
NS = 16
LANES = 16
NW = NC * NS
ROWS_PER_W = BATCH // NW
VOCAB_N = 100000
SLICE = 6256


def _sc_kernel(xt_hbm, table_hbm, bias_hbm, out_hbm, spt, idx_v, vals_v,
               out_v, bias_v, tab_v, sem):
    cid = lax.axis_index("c")
    sid = lax.axis_index("s")
    wid = sid * NC + cid
    base = wid * ROWS_PER_W

    stage = [
        pltpu.async_copy(
            xt_hbm.at[f, pl.ds(base, ROWS_PER_W)],
            idx_v.at[pl.ds(f * ROWS_PER_W, ROWS_PER_W)],
            sem,
        )
        for f in range(NUM_FIELDS)
    ]
    bias_cp = pltpu.async_copy(bias_hbm, bias_v, sem)

    slice_start = sid * SLICE
    last_start = (NS - 1) * SLICE

    @pl.when(sid != NS - 1)
    def _copy_full():
        pltpu.sync_copy(table_hbm.at[pl.ds(slice_start, SLICE)], tab_v)
        pltpu.sync_copy(tab_v, spt.at[pl.ds(slice_start, SLICE)])

    @pl.when(sid == NS - 1)
    def _copy_tail():
        n = VOCAB_N - last_start
        pltpu.sync_copy(table_hbm.at[pl.ds(last_start, n)],
                        tab_v.at[pl.ds(0, n)])
        pltpu.sync_copy(tab_v.at[pl.ds(0, n)],
                        spt.at[pl.ds(last_start, n)])

    plsc.subcore_barrier()

    for cp in stage:
        cp.wait()

    pltpu.async_copy(spt.at[idx_v], vals_v, sem).wait()

    bias_cp.wait()
    bias_vec = plsc.load_gather(bias_v, [jnp.zeros((LANES,), jnp.int32)])

    for j in range(ROWS_PER_W // LANES):
        acc = bias_vec
        for f in range(NUM_FIELDS):
            acc = acc + vals_v[pl.ds(f * ROWS_PER_W + j * LANES, LANES)]
        out_v[pl.ds(j * LANES, LANES)] = acc

    pltpu.sync_copy(out_v, out_hbm.at[pl.ds(base, ROWS_PER_W)])


@jax.jit
def _run(xt, table_flat, bias):
    mesh = plsc.VectorSubcoreMesh(
        core_axis_name="c", subcore_axis_name="s",
        num_cores=NC, num_subcores=NS)
    f = functools.partial(
        pl.kernel,
        out_type=jax.ShapeDtypeStruct((BATCH,), jnp.float32),
        mesh=mesh,
        scratch_types=[
            pltpu.VMEM_SHARED((NS * SLICE,), jnp.float32),
            pltpu.VMEM((NUM_FIELDS * ROWS_PER_W,), jnp.int32),
            pltpu.VMEM((NUM_FIELDS * ROWS_PER_W,), jnp.float32),
            pltpu.VMEM((ROWS_PER_W,), jnp.float32),
            pltpu.VMEM((1,), jnp.float32),
            pltpu.VMEM((SLICE,), jnp.float32),
            pltpu.SemaphoreType.DMA,
        ],
        compiler_params=pltpu.CompilerParams(needs_layout_passes=False),
    )(_sc_kernel)
    return f(xt, table_flat, bias)


def kernel(x, table, bias):
    xt = x.astype(jnp.int32).T
    table_flat = table.reshape(-1)
    out = _run(xt, table_flat, bias.astype(jnp.float32))
    return out.reshape(BATCH, 1)
